# baseline (speedup 1.0000x reference)
import functools

import jax
import jax.numpy as jnp
from jax import lax
from jax.experimental import pallas as pl
from jax.experimental.pallas import tpu as pltpu
from jax.experimental.pallas import tpu_sc as plsc
from jax.experimental import layout as jax_layout

NUM_CORES = 2
NUM_SUBCORES = 16
NUM_WORKERS = NUM_CORES * NUM_SUBCORES
NBUF = 4


def _make_lookup(batch: int, text: int, dim: int, rows_per_chunk: int):
  assert batch % NUM_WORKERS == 0
  rows_per_w = batch // NUM_WORKERS
  assert rows_per_w % (NBUF * rows_per_chunk) == 0
  chunk = rows_per_chunk * text
  n_chunks = rows_per_w // rows_per_chunk
  n_groups = n_chunks // NBUF
  idx_per_w = rows_per_w * text
  assert chunk % 8 == 0

  mesh = plsc.VectorSubcoreMesh(core_axis_name="c", subcore_axis_name="s")

  @functools.partial(
      pl.kernel,
      mesh=mesh,
      out_type=jax.ShapeDtypeStruct((batch, text, dim), jnp.float32),
      scratch_types=[
          pltpu.VMEM((idx_per_w,), jnp.int32),
          [pltpu.VMEM((chunk, dim), jnp.float32) for _ in range(NBUF)],
          [pltpu.SemaphoreType.DMA for _ in range(NBUF)],
      ],
  )
  def lookup_kernel(table_hbm, idx_hbm, out_hbm, idx_v, bufs, sems):
    wid = lax.axis_index("s") * NUM_CORES + lax.axis_index("c")
    row_base = wid * rows_per_w
    pltpu.sync_copy(idx_hbm.at[pl.ds(row_base * text, idx_per_w)], idx_v)

    def gather_start(c, j):
      pltpu.async_copy(
          table_hbm.at[idx_v.at[pl.ds(c * chunk, chunk)]], bufs[j], sems[j]
      )

    def gather_wait(c, j):
      pltpu.make_async_copy(
          table_hbm.at[idx_v.at[pl.ds(c * chunk, chunk)]], bufs[j], sems[j]
      ).wait()

    def store(c, j):
      row0 = row_base + c * rows_per_chunk
      for r in range(rows_per_chunk):
        pltpu.sync_copy(
            bufs[j].at[pl.ds(r * text, text)], out_hbm.at[row0 + r]
        )

    for j in range(NBUF - 1):
      gather_start(j, j)

    def body(g, carry):
      c0 = g * NBUF
      for j in range(NBUF):
        c = c0 + j
        nxt = c + NBUF - 1
        jn = (j + NBUF - 1) % NBUF
        @pl.when(nxt < n_chunks)
        def _(nxt=nxt, jn=jn):
          gather_start(nxt, jn)
        gather_wait(c, j)
        store(c, j)
      return carry

    lax.fori_loop(0, n_groups, body, 0)

  return lookup_kernel


_NSPLIT = 4
_PART_B = 4096 // _NSPLIT
_lookup_part = _make_lookup(_PART_B, 50, 128, 4)

def kernel(input, table):
  idx = input.astype(jnp.int32)
  tiled = jax_layout.Layout(major_to_minor=(0, 1, 2), tiling=((8, 128),))
  acc = jax_layout.with_layout_constraint(
      jnp.zeros((4096, 50, 128), jnp.float32), tiled
  )
  for k in range(_NSPLIT):
    part = _lookup_part(table, idx[k * _PART_B:(k + 1) * _PART_B].reshape(-1))
    acc = lax.dynamic_update_slice(acc, part, (k * _PART_B, 0, 0))
  return acc


# speedup vs baseline: 1.7486x; 1.7486x over previous
import functools

import jax
import jax.numpy as jnp
from jax import lax
from jax.experimental import pallas as pl
from jax.experimental.pallas import tpu as pltpu
from jax.experimental.pallas import tpu_sc as plsc

NUM_CORES = 2
NUM_SUBCORES = 16
NUM_WORKERS = NUM_CORES * NUM_SUBCORES
NBUF = 4


def _make_lookup(batch: int, text: int, dim: int, rows_per_chunk: int):
  assert batch % NUM_WORKERS == 0
  rows_per_w = batch // NUM_WORKERS
  assert rows_per_w % (NBUF * rows_per_chunk) == 0
  chunk = rows_per_chunk * text
  n_chunks = rows_per_w // rows_per_chunk
  n_groups = n_chunks // NBUF
  idx_per_w = rows_per_w * text
  assert chunk % 8 == 0

  mesh = plsc.VectorSubcoreMesh(core_axis_name="c", subcore_axis_name="s")

  @functools.partial(
      pl.kernel,
      mesh=mesh,
      out_type=jax.ShapeDtypeStruct((batch, text, dim), jnp.float32),
      scratch_types=[
          pltpu.VMEM((idx_per_w,), jnp.int32),
          [pltpu.VMEM((chunk, dim), jnp.float32) for _ in range(NBUF)],
          [pltpu.SemaphoreType.DMA for _ in range(NBUF)],
      ],
  )
  def lookup_kernel(table_hbm, idx_hbm, out_hbm, idx_v, bufs, sems):
    wid = lax.axis_index("s") * NUM_CORES + lax.axis_index("c")
    row_base = wid * rows_per_w
    pltpu.sync_copy(idx_hbm.at[pl.ds(row_base * text, idx_per_w)], idx_v)

    def gather_start(c, j):
      pltpu.async_copy(
          table_hbm.at[idx_v.at[pl.ds(c * chunk, chunk)]], bufs[j], sems[j]
      )

    def gather_wait(c, j):
      pltpu.make_async_copy(
          table_hbm.at[idx_v.at[pl.ds(c * chunk, chunk)]], bufs[j], sems[j]
      ).wait()

    def store(c, j):
      row0 = row_base + c * rows_per_chunk
      for r in range(rows_per_chunk):
        pltpu.sync_copy(
            bufs[j].at[pl.ds(r * text, text)], out_hbm.at[row0 + r]
        )

    for j in range(NBUF - 1):
      gather_start(j, j)

    def body(g, carry):
      c0 = g * NBUF
      for j in range(NBUF):
        c = c0 + j
        nxt = c + NBUF - 1
        jn = (j + NBUF - 1) % NBUF
        @pl.when(nxt < n_chunks)
        def _(nxt=nxt, jn=jn):
          gather_start(nxt, jn)
        gather_wait(c, j)
        store(c, j)
      return carry

    lax.fori_loop(0, n_groups, body, 0)

  return lookup_kernel


_lookup = _make_lookup(4096, 50, 128, 4)


def kernel(input, table):
  idx = input.reshape(-1).astype(jnp.int32)
  return _lookup(table, idx)
